# Initial kernel scaffold; baseline (speedup 1.0000x reference)
import jax
import jax.numpy as jnp
from jax.experimental import pallas as pl


def kernel(scores):
    raise NotImplementedError("write your pallas kernel here")


# speedup vs baseline: 67.6397x; 67.6397x over previous
import functools

import jax
import jax.numpy as jnp
from jax import lax
from jax.experimental import pallas as pl
from jax.experimental.pallas import tpu as pltpu
from jax.experimental.pallas import tpu_sc as plsc

_B, _N = 32, 1024
_R = 256
_L = 16
_V = _N // _L
_D0 = 16


def _sc_sort_body(scores_hbm, sorted_hbm, d_hbm, xb, yb, ob, db):
    wid = lax.axis_index("s") * 2 + lax.axis_index("c")
    pltpu.sync_copy(scores_hbm.at[wid], xb.at[pl.ds(_D0, _N)])

    def _init(i, _):
        off = _D0 + i * _L
        v = -xb[pl.ds(off, _L)]
        sv, _ = plsc.sort_key_val(v, v)
        xb[pl.ds(off, _L)] = sv
        return 0

    lax.fori_loop(0, _V, _init, 0)

    m = _L
    while m < _N:
        k = m // _L

        def _pair(p, _, m=m, k=k):
            base = _D0 + p * 2 * m
            for i in range(k):
                a = xb[pl.ds(base + i * _L, _L)]
                b = xb[pl.ds(base + m + (k - 1 - i) * _L, _L)]
                rb = lax.rev(b, (0,))
                yb[pl.ds(base + i * _L, _L)] = jnp.minimum(a, rb)
                yb[pl.ds(base + m + i * _L, _L)] = jnp.maximum(a, rb)
            dd = k // 2
            while dd >= 1:
                for g in range(k // dd):
                    for i in range(dd):
                        u = base + (g * 2 * dd + i) * _L
                        w = u + dd * _L
                        av = yb[pl.ds(u, _L)]
                        bv = yb[pl.ds(w, _L)]
                        yb[pl.ds(u, _L)] = jnp.minimum(av, bv)
                        yb[pl.ds(w, _L)] = jnp.maximum(av, bv)
                dd //= 2
            for i in range(2 * k):
                v = yb[pl.ds(base + i * _L, _L)]
                sv, _ = plsc.sort_key_val(v, v)
                xb[pl.ds(base + i * _L, _L)] = sv
            return 0

        lax.fori_loop(0, _N // (2 * m), _pair, 0)
        m *= 2

    xb[pl.ds(_D0 - _L, _L)] = jnp.full((_L,), -jnp.inf, jnp.float32)
    xb[pl.ds(_D0 + _N, _L)] = jnp.full((_L,), jnp.inf, jnp.float32)

    def _fin(i, _):
        off = _D0 + i * _L
        cur = xb[pl.ds(off, _L)]
        prv = xb[pl.ds(off - 1, _L)]
        nxt = xb[pl.ds(off + 1, _L)]
        ob[pl.ds(i * _L, _L)] = -cur
        db[pl.ds(i * _L, _L)] = jnp.minimum(cur - prv, nxt - cur)
        return 0

    lax.fori_loop(0, _V, _fin, 0)
    pltpu.sync_copy(ob, sorted_hbm.at[wid])
    pltpu.sync_copy(db, d_hbm.at[wid])


_sc_sort = functools.partial(
    pl.kernel,
    out_type=[
        jax.ShapeDtypeStruct((_B, _N), jnp.float32),
        jax.ShapeDtypeStruct((_B, _N), jnp.float32),
    ],
    mesh=plsc.VectorSubcoreMesh(core_axis_name="c", subcore_axis_name="s"),
    compiler_params=pltpu.CompilerParams(
        needs_layout_passes=False, use_tc_tiling_on_sc=False),
    scratch_types=[
        pltpu.VMEM((_D0 + _N + _L,), jnp.float32),
        pltpu.VMEM((_D0 + _N + _L,), jnp.float32),
        pltpu.VMEM((_N,), jnp.float32),
        pltpu.VMEM((_N,), jnp.float32),
    ],
)(_sc_sort_body)


def _build_block(scores_ref, sd_ref, out_ref):
    x = scores_ref[0]
    s = sd_ref[0, :, 0:1]
    d = sd_ref[0, :, 1:2]
    half = d * 0.5
    p = -jnp.abs(x - s)
    num = jnp.maximum(p + half, 0.0)
    out_ref[...] = (num / half)[None]


def kernel(scores):
    sorted_desc, d = _sc_sort(scores)
    sd = jnp.stack([sorted_desc, d], axis=-1)
    return pl.pallas_call(
        _build_block,
        grid=(_B, _N // _R),
        in_specs=[
            pl.BlockSpec((1, 1, _N), lambda b, r: (b, 0, 0)),
            pl.BlockSpec((1, _R, 2), lambda b, r: (b, r, 0)),
        ],
        out_specs=pl.BlockSpec((1, _R, _N), lambda b, r: (b, r, 0)),
        out_shape=jax.ShapeDtypeStruct((_B, _N, _N), scores.dtype),
    )(scores[:, None, :], sd)
